# baseline (speedup 1.0000x reference)
import jax
import jax.numpy as jnp
from jax.experimental import pallas as pl


def _copy_body(x_ref, o_ref):
    o_ref[...] = x_ref[...]


def kernel(input):
    b, c, h, w = input.shape
    return pl.pallas_call(
        _copy_body,
        grid=(c,),
        in_specs=[pl.BlockSpec((b, 1, h, w), lambda j: (0, c - 1 - j, 0, 0))],
        out_specs=pl.BlockSpec((b, 1, h, w), lambda j: (0, j, 0, 0)),
        out_shape=jax.ShapeDtypeStruct((b, c, h, w), input.dtype),
    )(input)


# speedup vs baseline: 1.9097x; 1.9097x over previous
import jax
import jax.numpy as jnp
from jax.experimental import pallas as pl

CB = 64


def _rev_body(x_ref, o_ref):
    for k in range(CB):
        o_ref[0, k] = x_ref[0, CB - 1 - k]


def kernel(input):
    b, c, h, w = input.shape
    nblk = c // CB
    return pl.pallas_call(
        _rev_body,
        grid=(b, nblk),
        in_specs=[
            pl.BlockSpec((1, CB, h, w), lambda i, j: (i, nblk - 1 - j, 0, 0))
        ],
        out_specs=pl.BlockSpec((1, CB, h, w), lambda i, j: (i, j, 0, 0)),
        out_shape=jax.ShapeDtypeStruct((b, c, h, w), input.dtype),
    )(input)


# speedup vs baseline: 3.5558x; 1.8620x over previous
import jax
import jax.numpy as jnp
from jax.experimental import pallas as pl

CB = 384


def _rev_body(x_ref, o_ref):
    r = jax.lax.broadcasted_iota(jnp.int32, (CB, CB), 0)
    c = jax.lax.broadcasted_iota(jnp.int32, (CB, CB), 1)
    p = (r + c == CB - 1).astype(jnp.float32)
    o_ref[...] = jax.lax.dot(
        p, x_ref[0], preferred_element_type=jnp.float32
    )[None]


def kernel(input):
    b, c, h, w = input.shape
    x = input.reshape(b, c, h * w)
    nblk = c // CB
    out = pl.pallas_call(
        _rev_body,
        grid=(b, nblk),
        in_specs=[pl.BlockSpec((1, CB, h * w), lambda i, j: (i, nblk - 1 - j, 0))],
        out_specs=pl.BlockSpec((1, CB, h * w), lambda i, j: (i, j, 0)),
        out_shape=jax.ShapeDtypeStruct((b, c, h * w), input.dtype),
    )(x)
    return out.reshape(b, c, h, w)
